# baseline (speedup 1.0000x reference)
import functools

import jax
import jax.numpy as jnp
from jax import lax
from jax.experimental import pallas as pl
from jax.experimental.pallas import tpu as pltpu
from jax.experimental.pallas import tpu_sc as plsc

POINTS = 100000
EMB_DIM = 32
CTX = 20
BATCH = 1024
HID = 128

BN = 1024
NBLK = pl.cdiv(POINTS, BN)

NC, NS = 2, 16
NW = NC * NS
TOTAL_ROWS = BATCH * CTX
ROWS_PER_W = TOTAL_ROWS // NW

NEG = -1e30


def _sc_gather(emb, idx):
    mesh = plsc.VectorSubcoreMesh(core_axis_name="c", subcore_axis_name="s")

    @functools.partial(
        pl.kernel,
        mesh=mesh,
        out_type=jax.ShapeDtypeStruct((TOTAL_ROWS, EMB_DIM), jnp.float32),
        scratch_types=[
            pltpu.VMEM((ROWS_PER_W,), jnp.int32),
            pltpu.VMEM((ROWS_PER_W, EMB_DIM), jnp.float32),
            pltpu.SemaphoreType.DMA,
        ],
        compiler_params=pltpu.CompilerParams(use_tc_tiling_on_sc=False),
    )
    def k(table_hbm, idx_hbm, out_hbm, idx_v, rows_v, sem):
        wid = lax.axis_index("s") * NC + lax.axis_index("c")
        base = wid * ROWS_PER_W
        pltpu.sync_copy(idx_hbm.at[pl.ds(base, ROWS_PER_W)], idx_v)
        pltpu.async_copy(table_hbm.at[idx_v], rows_v, sem).wait()
        pltpu.sync_copy(rows_v, out_hbm.at[pl.ds(base, ROWS_PER_W)])

    return k(emb, idx)


def _lse_body(x_ref, w1t_ref, b1_ref, w2_ref, b2_ref, h_out, lse_out,
              m_ref, s_ref):
    j = pl.program_id(0)

    @pl.when(j == 0)
    def _init():
        h = jnp.dot(x_ref[...], w1t_ref[...],
                    preferred_element_type=jnp.float32) + b1_ref[...]
        h_out[...] = jnp.maximum(h, 0.0)
        m_ref[...] = jnp.full(m_ref.shape, NEG, jnp.float32)
        s_ref[...] = jnp.zeros(s_ref.shape, jnp.float32)

    logits = lax.dot_general(h_out[...], w2_ref[...],
                             (((1,), (1,)), ((), ())),
                             preferred_element_type=jnp.float32) + b2_ref[...]
    cols = j * BN + lax.broadcasted_iota(jnp.int32, logits.shape, 1)
    logits = jnp.where(cols < POINTS, logits, NEG)
    bm = jnp.max(logits, axis=1, keepdims=True)
    m_old = m_ref[...]
    m_new = jnp.maximum(m_old, bm)
    s_ref[...] = (s_ref[...] * jnp.exp(m_old - m_new)
                  + jnp.sum(jnp.exp(logits - m_new), axis=1, keepdims=True))
    m_ref[...] = m_new

    @pl.when(j == NBLK - 1)
    def _fin():
        lse_out[...] = m_ref[...] + jnp.log(s_ref[...])


def _write_body(h_ref, lse_ref, w2_ref, b2_ref, o_ref):
    logits = lax.dot_general(h_ref[...], w2_ref[...],
                             (((1,), (1,)), ((), ())),
                             preferred_element_type=jnp.float32)
    o_ref[...] = logits + b2_ref[...] - lse_ref[...]


def _lse_call(x, w1t, b1r, W2, b2r, interpret=False):
    return pl.pallas_call(
        _lse_body,
        grid=(NBLK,),
        in_specs=[
            pl.BlockSpec((BATCH, EMB_DIM * CTX), lambda j: (0, 0)),
            pl.BlockSpec((EMB_DIM * CTX, HID), lambda j: (0, 0)),
            pl.BlockSpec((1, HID), lambda j: (0, 0)),
            pl.BlockSpec((BN, HID), lambda j: (j, 0)),
            pl.BlockSpec((1, BN), lambda j: (0, j)),
        ],
        out_specs=[
            pl.BlockSpec((BATCH, HID), lambda j: (0, 0)),
            pl.BlockSpec((BATCH, 1), lambda j: (0, 0)),
        ],
        out_shape=[
            jax.ShapeDtypeStruct((BATCH, HID), jnp.float32),
            jax.ShapeDtypeStruct((BATCH, 1), jnp.float32),
        ],
        scratch_shapes=[
            pltpu.VMEM((BATCH, 1), jnp.float32),
            pltpu.VMEM((BATCH, 1), jnp.float32),
        ],
        compiler_params=pltpu.CompilerParams(
            dimension_semantics=("arbitrary",)),
        interpret=interpret,
    )(x, w1t, b1r, W2, b2r)


def _write_call(h, lse, W2, b2r, interpret=False):
    return pl.pallas_call(
        _write_body,
        grid=(NBLK,),
        in_specs=[
            pl.BlockSpec((BATCH, HID), lambda j: (0, 0)),
            pl.BlockSpec((BATCH, 1), lambda j: (0, 0)),
            pl.BlockSpec((BN, HID), lambda j: (j, 0)),
            pl.BlockSpec((1, BN), lambda j: (0, j)),
        ],
        out_specs=pl.BlockSpec((BATCH, BN), lambda j: (0, j)),
        out_shape=jax.ShapeDtypeStruct((BATCH, POINTS), jnp.float32),
        compiler_params=pltpu.CompilerParams(
            dimension_semantics=("arbitrary",)),
        interpret=interpret,
    )(h, lse, W2, b2r)


def kernel(inputs, emb, W1, b1, W2, b2):
    idx = inputs.reshape(-1).astype(jnp.int32)
    x = _sc_gather(emb, idx).reshape(BATCH, EMB_DIM * CTX)
    w1t = W1.T
    b1r = b1.reshape(1, HID)
    b2r = b2.reshape(1, POINTS)
    h, lse = _lse_call(x, w1t, b1r, W2, b2r)
    return _write_call(h, lse, W2, b2r)
